# baseline (speedup 1.0000x reference)
import functools
import jax
import jax.numpy as jnp
from jax import lax
from jax.experimental import pallas as pl
from jax.experimental.pallas import tpu as pltpu
from jax.experimental.pallas import tpu_sc as plsc

_N = 10000
_E = 320000
_D = 128
_NC = 2
_NS = 16
_NW = _NC * _NS
_CH = 128
_RPW = 80
_EPW = _RPW * _CH
_EPAD = _NW * _EPW
_RPT = 632
_ACC_ROWS = _NS * _RPT
_DUMP_ROW = _N + 16


def _matmul(x, w):
    def mm(x_ref, w_ref, o_ref):
        o_ref[...] = jnp.dot(x_ref[...], w_ref[...],
                             preferred_element_type=jnp.float32)
    return pl.pallas_call(
        mm,
        grid=(10,),
        in_specs=[pl.BlockSpec((_N // 10, _D), lambda i: (i, 0)),
                  pl.BlockSpec((_D, _D), lambda i: (0, 0))],
        out_specs=pl.BlockSpec((_N // 10, _D), lambda i: (i, 0)),
        out_shape=jax.ShapeDtypeStruct((_N, _D), jnp.float32),
    )(x, w)


def _add_partials(parts):
    def body(p_ref, o_ref):
        o_ref[...] = p_ref[0] + p_ref[1]
    return pl.pallas_call(
        body,
        grid=(10,),
        in_specs=[pl.BlockSpec((2, _N // 10, _D), lambda i: (0, i, 0))],
        out_specs=pl.BlockSpec((_N // 10, _D), lambda i: (i, 0)),
        out_shape=jax.ShapeDtypeStruct((_N, _D), jnp.float32),
    )(parts)


@functools.partial(
    pl.kernel,
    out_type=jax.ShapeDtypeStruct((_NC * _ACC_ROWS, _D), jnp.float32),
    mesh=plsc.VectorSubcoreMesh(core_axis_name="c", subcore_axis_name="s"),
    scratch_types=[
        [pltpu.VMEM((_CH,), jnp.int32)] * 4,
        [pltpu.VMEM((_CH,), jnp.int32)] * 4,
        [pltpu.VMEM((_CH, _D), jnp.float32)] * 2,
        pltpu.VMEM_SHARED((_ACC_ROWS, _D), jnp.float32),
        [pltpu.SemaphoreType.DMA] * 2,
        [pltpu.SemaphoreType.DMA] * 4,
    ],
)
def _spmm_round(xp_hbm, col_hbm, rid_hbm, out_hbm,
                colb, ridb, rows, acc, semg, semi):
    cid = lax.axis_index("c")
    sid = lax.axis_index("s")
    wid = cid * _NS + sid
    ebase = wid * _EPW

    def zstore(i, _):
        for j in range(_D // 16):
            rows[0][i, pl.ds(j * 16, 16)] = jnp.zeros((16,), jnp.float32)
        return 0
    lax.fori_loop(0, _CH, zstore, 0)

    zbase = sid * _RPT
    def zcopy(k, _):
        pltpu.sync_copy(rows[0], acc.at[pl.ds(zbase + k * _CH, _CH)])
        return 0
    lax.fori_loop(0, _RPT // _CH, zcopy, 0)
    pltpu.sync_copy(rows[0].at[pl.ds(0, _RPT % _CH)],
                    acc.at[pl.ds(zbase + (_RPT // _CH) * _CH, _RPT % _CH)])

    def idxload(k, i):
        off = ebase + k * _CH
        pltpu.async_copy(col_hbm.at[pl.ds(off, _CH)], colb[i], semi[i])
        pltpu.async_copy(rid_hbm.at[pl.ds(off, _CH)], ridb[i], semi[i])

    def idxwait(i):
        pltpu.make_async_copy(col_hbm.at[pl.ds(0, _CH)], colb[i],
                              semi[i]).wait()
        pltpu.make_async_copy(rid_hbm.at[pl.ds(0, _CH)], ridb[i],
                              semi[i]).wait()

    def gather(i, r):
        pltpu.async_copy(xp_hbm.at[colb[i]], rows[r], semg[r])

    def gwait(i, r):
        pltpu.make_async_copy(xp_hbm.at[colb[i]], rows[r], semg[r]).wait()

    def scatter(i, r):
        pltpu.sync_copy(rows[r], acc.at[ridb[i]], add=True)

    for i in range(4):
        idxload(i, i)

    plsc.subcore_barrier()

    idxwait(0)
    gather(0, 0)

    def step(j, _):
        a = 4 * j
        idxwait(1)
        gather(1, 1)
        gwait(0, 0)
        scatter(0, 0)

        @pl.when(a + 4 < _RPW)
        def _():
            idxload(a + 4, 0)

        idxwait(2)
        gather(2, 0)
        gwait(1, 1)
        scatter(1, 1)

        @pl.when(a + 5 < _RPW)
        def _():
            idxload(a + 5, 1)

        idxwait(3)
        gather(3, 1)
        gwait(2, 0)
        scatter(2, 0)

        @pl.when(a + 6 < _RPW)
        def _():
            idxload(a + 6, 2)

        @pl.when(a + 4 < _RPW)
        def _():
            idxwait(0)
            gather(0, 0)
        gwait(3, 1)
        scatter(3, 1)

        @pl.when(a + 7 < _RPW)
        def _():
            idxload(a + 7, 3)
        return 0
    lax.fori_loop(0, _RPW // 4, step, 0)

    plsc.subcore_barrier()

    pltpu.sync_copy(acc.at[pl.ds(zbase, _RPT)],
                    out_hbm.at[pl.ds(cid * _ACC_ROWS + zbase, _RPT)])


def kernel(X, row_pointers, column_index, blockPartition, edgeToColumn,
           edgeToRow, W, attention_w):
    deg = row_pointers[1:] - row_pointers[:-1]
    row_ids = jnp.repeat(jnp.arange(_N, dtype=jnp.int32), deg,
                         total_repeat_length=_E)
    pad = _EPAD - _E
    col_p = jnp.concatenate([column_index, jnp.zeros((pad,), jnp.int32)])
    rid_p = jnp.concatenate([row_ids, jnp.full((pad,), _DUMP_ROW, jnp.int32)])
    xp = _matmul(X, W)
    for _ in range(8):
        flat = _spmm_round(xp, col_p, rid_p)
        xp = _add_partials(flat.reshape(_NC, _ACC_ROWS, _D))
    return xp


# speedup vs baseline: 1.0812x; 1.0812x over previous
import functools
import jax
import jax.numpy as jnp
from jax import lax
from jax.experimental import pallas as pl
from jax.experimental.pallas import tpu as pltpu
from jax.experimental.pallas import tpu_sc as plsc

_N = 10000
_E = 320000
_D = 128
_NC = 2
_NS = 16
_NW = _NC * _NS
_CH = 128
_RPW = 80
_EPW = _RPW * _CH
_EPAD = _NW * _EPW
_RPT = 632
_ACC_ROWS = _NS * _RPT
_DUMP_ROW = _N + 16
_RP_PAD = 16384


def _matmul(x, w):
    def mm(x_ref, w_ref, o_ref):
        o_ref[...] = jnp.dot(x_ref[...], w_ref[...],
                             preferred_element_type=jnp.float32)
    return pl.pallas_call(
        mm,
        grid=(10,),
        in_specs=[pl.BlockSpec((_N // 10, _D), lambda i: (i, 0)),
                  pl.BlockSpec((_D, _D), lambda i: (0, 0))],
        out_specs=pl.BlockSpec((_N // 10, _D), lambda i: (i, 0)),
        out_shape=jax.ShapeDtypeStruct((_N, _D), jnp.float32),
    )(x, w)


def _add_partials(parts):
    def body(p_ref, o_ref):
        o_ref[...] = p_ref[0] + p_ref[1]
    return pl.pallas_call(
        body,
        grid=(10,),
        in_specs=[pl.BlockSpec((2, _N // 10, _D), lambda i: (0, i, 0))],
        out_specs=pl.BlockSpec((_N // 10, _D), lambda i: (i, 0)),
        out_shape=jax.ShapeDtypeStruct((_N, _D), jnp.float32),
    )(parts)


@functools.partial(
    pl.kernel,
    out_type=jax.ShapeDtypeStruct((_EPAD,), jnp.int32),
    mesh=plsc.VectorSubcoreMesh(core_axis_name="c", subcore_axis_name="s"),
    compiler_params=pltpu.CompilerParams(needs_layout_passes=False),
    scratch_types=[
        pltpu.VMEM((_RP_PAD,), jnp.int32),
        pltpu.VMEM((_EPW,), jnp.int32),
    ],
)
def _build_rids(rp_hbm, out_hbm, rpv, ridv):
    cid = lax.axis_index("c")
    sid = lax.axis_index("s")
    wid = cid * _NS + sid
    ebase = wid * _EPW

    big = jnp.full((16,), jnp.iinfo(jnp.int32).max, jnp.int32)
    def fillmax(i, _):
        rpv[pl.ds(_N + i * 16, 16)] = big
        return 0
    lax.fori_loop(0, (_RP_PAD - _N) // 16, fillmax, 0)
    pltpu.sync_copy(rp_hbm.at[pl.ds(0, _N)], rpv.at[pl.ds(0, _N)])

    iota = lax.iota(jnp.int32, 16)
    dumpv = jnp.full((16,), _DUMP_ROW, jnp.int32)
    ev_max = jnp.full((16,), _E, jnp.int32)
    def vec(v, _):
        ev = iota + (ebase + v * 16)
        lo = jnp.zeros((16,), jnp.int32)
        w = _RP_PAD // 2
        while w >= 1:
            cand = lo + jnp.full((16,), w, jnp.int32)
            vals = plsc.load_gather(rpv, [cand])
            lo = jnp.where(vals <= ev, cand, lo)
            w //= 2
        ridv[pl.ds(v * 16, 16)] = jnp.where(ev < ev_max, lo, dumpv)
        return 0
    lax.fori_loop(0, _EPW // 16, vec, 0)

    pltpu.sync_copy(ridv, out_hbm.at[pl.ds(ebase, _EPW)])


@functools.partial(
    pl.kernel,
    out_type=jax.ShapeDtypeStruct((_NC * _ACC_ROWS, _D), jnp.float32),
    mesh=plsc.VectorSubcoreMesh(core_axis_name="c", subcore_axis_name="s"),
    scratch_types=[
        pltpu.VMEM((_CH,), jnp.int32),
        pltpu.VMEM((_CH,), jnp.int32),
        pltpu.VMEM((_CH, _D), jnp.float32),
        pltpu.VMEM_SHARED((_ACC_ROWS, _D), jnp.float32),
        pltpu.SemaphoreType.DMA,
    ],
)
def _spmm_round(xp_hbm, col_hbm, rid_hbm, out_hbm,
                colv, ridv, rowsv, acc, sem):
    cid = lax.axis_index("c")
    sid = lax.axis_index("s")
    wid = cid * _NS + sid
    ebase = wid * _EPW

    def zstore(i, _):
        for j in range(_D // 16):
            rowsv[i, pl.ds(j * 16, 16)] = jnp.zeros((16,), jnp.float32)
        return 0
    lax.fori_loop(0, _CH, zstore, 0)

    zbase = sid * _RPT
    def zcopy(k, _):
        pltpu.sync_copy(rowsv, acc.at[pl.ds(zbase + k * _CH, _CH)])
        return 0
    lax.fori_loop(0, _RPT // _CH, zcopy, 0)
    pltpu.sync_copy(rowsv.at[pl.ds(0, _RPT % _CH)],
                    acc.at[pl.ds(zbase + (_RPT // _CH) * _CH, _RPT % _CH)])

    plsc.subcore_barrier()

    def step(k, _):
        off = ebase + k * _CH
        pltpu.sync_copy(col_hbm.at[pl.ds(off, _CH)], colv)
        pltpu.sync_copy(rid_hbm.at[pl.ds(off, _CH)], ridv)
        pltpu.async_copy(xp_hbm.at[colv], rowsv, sem).wait()
        pltpu.sync_copy(rowsv, acc.at[ridv], add=True)
        return 0
    lax.fori_loop(0, _RPW, step, 0)

    plsc.subcore_barrier()

    pltpu.sync_copy(acc.at[pl.ds(zbase, _RPT)],
                    out_hbm.at[pl.ds(cid * _ACC_ROWS + zbase, _RPT)])


def kernel(X, row_pointers, column_index, blockPartition, edgeToColumn,
           edgeToRow, W, attention_w):
    pad = _EPAD - _E
    col_p = jnp.concatenate([column_index, jnp.zeros((pad,), jnp.int32)])
    rid_p = _build_rids(row_pointers)
    xp = _matmul(X, W)
    for _ in range(8):
        flat = _spmm_round(xp, col_p, rid_p)
        xp = _add_partials(flat.reshape(_NC, _ACC_ROWS, _D))
    return xp


# speedup vs baseline: 2.6159x; 2.4195x over previous
import functools
import jax
import jax.numpy as jnp
from jax import lax
from jax.experimental import pallas as pl
from jax.experimental.pallas import tpu as pltpu
from jax.experimental.pallas import tpu_sc as plsc

_N = 10000
_E = 320000
_D = 128
_NC = 2
_NS = 16
_NW = _NC * _NS
_CH = 128
_RPW = 80
_EPW = _RPW * _CH
_EPAD = _NW * _EPW
_RPT = 632
_ACC_ROWS = _NS * _RPT
_DUMP_ROW = _N + 16
_RP_PAD = 16384


def _matmul(x, w):
    def mm(x_ref, w_ref, o_ref):
        o_ref[...] = jnp.dot(x_ref[...], w_ref[...],
                             preferred_element_type=jnp.float32)
    return pl.pallas_call(
        mm,
        grid=(10,),
        in_specs=[pl.BlockSpec((_N // 10, _D), lambda i: (i, 0)),
                  pl.BlockSpec((_D, _D), lambda i: (0, 0))],
        out_specs=pl.BlockSpec((_N // 10, _D), lambda i: (i, 0)),
        out_shape=jax.ShapeDtypeStruct((_N, _D), jnp.float32),
    )(x, w)


def _add_partials(parts):
    def body(p_ref, o_ref):
        o_ref[...] = p_ref[0] + p_ref[1]
    return pl.pallas_call(
        body,
        grid=(10,),
        in_specs=[pl.BlockSpec((2, _N // 10, _D), lambda i: (0, i, 0))],
        out_specs=pl.BlockSpec((_N // 10, _D), lambda i: (i, 0)),
        out_shape=jax.ShapeDtypeStruct((_N, _D), jnp.float32),
    )(parts)


@functools.partial(
    pl.kernel,
    out_type=jax.ShapeDtypeStruct((_EPAD,), jnp.int32),
    mesh=plsc.VectorSubcoreMesh(core_axis_name="c", subcore_axis_name="s"),
    compiler_params=pltpu.CompilerParams(needs_layout_passes=False),
    scratch_types=[
        pltpu.VMEM((_RP_PAD,), jnp.int32),
        pltpu.VMEM((_EPW,), jnp.int32),
    ],
)
def _build_rids(rp_hbm, out_hbm, rpv, ridv):
    cid = lax.axis_index("c")
    sid = lax.axis_index("s")
    wid = cid * _NS + sid
    ebase = wid * _EPW

    big = jnp.full((16,), jnp.iinfo(jnp.int32).max, jnp.int32)
    def fillmax(i, _):
        rpv[pl.ds(_N + i * 16, 16)] = big
        return 0
    lax.fori_loop(0, (_RP_PAD - _N) // 16, fillmax, 0)
    pltpu.sync_copy(rp_hbm.at[pl.ds(0, _N)], rpv.at[pl.ds(0, _N)])

    iota = lax.iota(jnp.int32, 16)
    dumpv = jnp.full((16,), _N, jnp.int32)
    ev_max = jnp.full((16,), _E, jnp.int32)
    def vec(v, _):
        ev = iota + (ebase + v * 16)
        lo = jnp.zeros((16,), jnp.int32)
        w = _RP_PAD // 2
        while w >= 1:
            cand = lo + jnp.full((16,), w, jnp.int32)
            vals = plsc.load_gather(rpv, [cand])
            lo = jnp.where(vals <= ev, cand, lo)
            w //= 2
        dump = dumpv + (ev & jnp.full((16,), 63, jnp.int32))
        ridv[pl.ds(v * 16, 16)] = jnp.where(ev < ev_max, lo, dump)
        return 0
    lax.fori_loop(0, _EPW // 16, vec, 0)

    pltpu.sync_copy(ridv, out_hbm.at[pl.ds(ebase, _EPW)])


@functools.partial(
    pl.kernel,
    out_type=jax.ShapeDtypeStruct((_NC * _ACC_ROWS, _D), jnp.float32),
    mesh=plsc.VectorSubcoreMesh(core_axis_name="c", subcore_axis_name="s"),
    scratch_types=[
        pltpu.VMEM((_CH,), jnp.int32),
        pltpu.VMEM((_CH,), jnp.int32),
        pltpu.VMEM((_CH, _D), jnp.float32),
        pltpu.VMEM_SHARED((_ACC_ROWS, _D), jnp.float32),
        pltpu.SemaphoreType.DMA,
    ],
)
def _spmm_round(xp_hbm, col_hbm, rid_hbm, out_hbm,
                colv, ridv, rowsv, acc, sem):
    cid = lax.axis_index("c")
    sid = lax.axis_index("s")
    wid = cid * _NS + sid
    ebase = wid * _EPW

    def zstore(i, _):
        for j in range(_D // 16):
            rowsv[i, pl.ds(j * 16, 16)] = jnp.zeros((16,), jnp.float32)
        return 0
    lax.fori_loop(0, _CH, zstore, 0)

    zbase = sid * _RPT
    def zcopy(k, _):
        pltpu.sync_copy(rowsv, acc.at[pl.ds(zbase + k * _CH, _CH)])
        return 0
    lax.fori_loop(0, _RPT // _CH, zcopy, 0)
    pltpu.sync_copy(rowsv.at[pl.ds(0, _RPT % _CH)],
                    acc.at[pl.ds(zbase + (_RPT // _CH) * _CH, _RPT % _CH)])

    plsc.subcore_barrier()

    def step(k, _):
        off = ebase + k * _CH
        pltpu.sync_copy(col_hbm.at[pl.ds(off, _CH)], colv)
        pltpu.sync_copy(rid_hbm.at[pl.ds(off, _CH)], ridv)
        pltpu.async_copy(xp_hbm.at[colv], rowsv, sem).wait()
        pltpu.sync_copy(rowsv, acc.at[ridv], add=True)
        return 0
    lax.fori_loop(0, _RPW, step, 0)

    plsc.subcore_barrier()

    pltpu.sync_copy(acc.at[pl.ds(zbase, _RPT)],
                    out_hbm.at[pl.ds(cid * _ACC_ROWS + zbase, _RPT)])


def kernel(X, row_pointers, column_index, blockPartition, edgeToColumn,
           edgeToRow, W, attention_w):
    pad = _EPAD - _E
    col_p = jnp.concatenate(
        [column_index, jnp.arange(pad, dtype=jnp.int32) % _N])
    rid_p = _build_rids(row_pointers)
    xp = _matmul(X, W)
    for _ in range(8):
        flat = _spmm_round(xp, col_p, rid_p)
        xp = _add_partials(flat.reshape(_NC, _ACC_ROWS, _D))
    return xp


# speedup vs baseline: 5.3347x; 2.0393x over previous
import functools
import jax
import jax.numpy as jnp
from jax import lax
from jax.experimental import pallas as pl
from jax.experimental.pallas import tpu as pltpu
from jax.experimental.pallas import tpu_sc as plsc

_N = 10000
_E = 320000
_D = 128
_NC = 2
_NS = 16
_NW = _NC * _NS
_CH = 128
_RPW = 80
_EPW = _RPW * _CH
_EPAD = _NW * _EPW
_RPT = 632
_ACC_ROWS = _NS * _RPT
_DUMP_ROW = _N + 16
_RP_PAD = 16384


def _matmul(x, w):
    def mm(x_ref, w_ref, o_ref):
        o_ref[...] = jnp.dot(x_ref[...], w_ref[...],
                             preferred_element_type=jnp.float32)
    return pl.pallas_call(
        mm,
        grid=(10,),
        in_specs=[pl.BlockSpec((_N // 10, _D), lambda i: (i, 0)),
                  pl.BlockSpec((_D, _D), lambda i: (0, 0))],
        out_specs=pl.BlockSpec((_N // 10, _D), lambda i: (i, 0)),
        out_shape=jax.ShapeDtypeStruct((_N, _D), jnp.float32),
    )(x, w)


def _add_partials(parts):
    def body(p_ref, o_ref):
        o_ref[...] = p_ref[0] + p_ref[1]
    return pl.pallas_call(
        body,
        grid=(10,),
        in_specs=[pl.BlockSpec((2, _N // 10, _D), lambda i: (0, i, 0))],
        out_specs=pl.BlockSpec((_N // 10, _D), lambda i: (i, 0)),
        out_shape=jax.ShapeDtypeStruct((_N, _D), jnp.float32),
    )(parts)


@functools.partial(
    pl.kernel,
    out_type=jax.ShapeDtypeStruct((_EPAD,), jnp.int32),
    mesh=plsc.VectorSubcoreMesh(core_axis_name="c", subcore_axis_name="s"),
    compiler_params=pltpu.CompilerParams(needs_layout_passes=False),
    scratch_types=[
        pltpu.VMEM((_RP_PAD,), jnp.int32),
        pltpu.VMEM((_EPW,), jnp.int32),
    ],
)
def _build_rids(rp_hbm, out_hbm, rpv, ridv):
    cid = lax.axis_index("c")
    sid = lax.axis_index("s")
    wid = cid * _NS + sid
    ebase = wid * _EPW

    big = jnp.full((16,), jnp.iinfo(jnp.int32).max, jnp.int32)
    def fillmax(i, _):
        rpv[pl.ds(_N + i * 16, 16)] = big
        return 0
    lax.fori_loop(0, (_RP_PAD - _N) // 16, fillmax, 0)
    pltpu.sync_copy(rp_hbm.at[pl.ds(0, _N)], rpv.at[pl.ds(0, _N)])

    iota = lax.iota(jnp.int32, 16)
    dumpv = jnp.full((16,), _N, jnp.int32)
    ev_max = jnp.full((16,), _E, jnp.int32)
    def vec(v, _):
        ev = iota + (ebase + v * 16)
        lo = jnp.zeros((16,), jnp.int32)
        w = _RP_PAD // 2
        while w >= 1:
            cand = lo + jnp.full((16,), w, jnp.int32)
            vals = plsc.load_gather(rpv, [cand])
            lo = jnp.where(vals <= ev, cand, lo)
            w //= 2
        dump = dumpv + (ev & jnp.full((16,), 63, jnp.int32))
        ridv[pl.ds(v * 16, 16)] = jnp.where(ev < ev_max, lo, dump)
        return 0
    lax.fori_loop(0, _EPW // 16, vec, 0)

    pltpu.sync_copy(ridv, out_hbm.at[pl.ds(ebase, _EPW)])


@functools.partial(
    pl.kernel,
    out_type=jax.ShapeDtypeStruct((_NC * _ACC_ROWS, _D), jnp.float32),
    mesh=plsc.VectorSubcoreMesh(core_axis_name="c", subcore_axis_name="s"),
    scratch_types=[
        [pltpu.VMEM((_CH,), jnp.int32)] * 4,
        [pltpu.VMEM((_CH,), jnp.int32)] * 4,
        [pltpu.VMEM((_CH, _D), jnp.float32)] * 2,
        pltpu.VMEM_SHARED((_ACC_ROWS, _D), jnp.float32),
        [pltpu.SemaphoreType.DMA] * 2,
        [pltpu.SemaphoreType.DMA] * 4,
    ],
)
def _spmm_round(xp_hbm, col_hbm, rid_hbm, out_hbm,
                colb, ridb, rows, acc, semg, semi):
    cid = lax.axis_index("c")
    sid = lax.axis_index("s")
    wid = cid * _NS + sid
    ebase = wid * _EPW

    def zstore(i, _):
        for j in range(_D // 16):
            rows[0][i, pl.ds(j * 16, 16)] = jnp.zeros((16,), jnp.float32)
        return 0
    lax.fori_loop(0, _CH, zstore, 0)

    zbase = sid * _RPT
    def zcopy(k, _):
        pltpu.sync_copy(rows[0], acc.at[pl.ds(zbase + k * _CH, _CH)])
        return 0
    lax.fori_loop(0, _RPT // _CH, zcopy, 0)
    pltpu.sync_copy(rows[0].at[pl.ds(0, _RPT % _CH)],
                    acc.at[pl.ds(zbase + (_RPT // _CH) * _CH, _RPT % _CH)])

    def idxload(k, i):
        off = ebase + k * _CH
        pltpu.async_copy(col_hbm.at[pl.ds(off, _CH)], colb[i], semi[i])
        pltpu.async_copy(rid_hbm.at[pl.ds(off, _CH)], ridb[i], semi[i])

    def idxwait(i):
        pltpu.make_async_copy(col_hbm.at[pl.ds(0, _CH)], colb[i],
                              semi[i]).wait()
        pltpu.make_async_copy(rid_hbm.at[pl.ds(0, _CH)], ridb[i],
                              semi[i]).wait()

    def gather(i, r):
        pltpu.async_copy(xp_hbm.at[colb[i]], rows[r], semg[r])

    def gwait(i, r):
        pltpu.make_async_copy(xp_hbm.at[colb[i]], rows[r], semg[r]).wait()

    def scatter(i, r):
        pltpu.sync_copy(rows[r], acc.at[ridb[i]], add=True)

    for i in range(4):
        idxload(i, i)

    plsc.subcore_barrier()

    idxwait(0)
    gather(0, 0)

    def step(j, _):
        a = 4 * j
        idxwait(1)
        gather(1, 1)
        gwait(0, 0)
        scatter(0, 0)

        @pl.when(a + 4 < _RPW)
        def _():
            idxload(a + 4, 0)

        idxwait(2)
        gather(2, 0)
        gwait(1, 1)
        scatter(1, 1)

        @pl.when(a + 5 < _RPW)
        def _():
            idxload(a + 5, 1)

        idxwait(3)
        gather(3, 1)
        gwait(2, 0)
        scatter(2, 0)

        @pl.when(a + 6 < _RPW)
        def _():
            idxload(a + 6, 2)

        @pl.when(a + 4 < _RPW)
        def _():
            idxwait(0)
            gather(0, 0)
        gwait(3, 1)
        scatter(3, 1)

        @pl.when(a + 7 < _RPW)
        def _():
            idxload(a + 7, 3)
        return 0
    lax.fori_loop(0, _RPW // 4, step, 0)

    plsc.subcore_barrier()

    pltpu.sync_copy(acc.at[pl.ds(zbase, _RPT)],
                    out_hbm.at[pl.ds(cid * _ACC_ROWS + zbase, _RPT)])


def kernel(X, row_pointers, column_index, blockPartition, edgeToColumn,
           edgeToRow, W, attention_w):
    pad = _EPAD - _E
    col_p = jnp.concatenate(
        [column_index, jnp.arange(pad, dtype=jnp.int32) % _N])
    rid_p = _build_rids(row_pointers)
    xp = _matmul(X, W)
    for _ in range(8):
        flat = _spmm_round(xp, col_p, rid_p)
        xp = _add_partials(flat.reshape(_NC, _ACC_ROWS, _D))
    return xp


# speedup vs baseline: 5.6252x; 1.0545x over previous
import functools
import jax
import jax.numpy as jnp
from jax import lax
from jax.experimental import pallas as pl
from jax.experimental.pallas import tpu as pltpu
from jax.experimental.pallas import tpu_sc as plsc

_N = 10000
_E = 320000
_D = 128
_NC = 2
_NS = 16
_NW = _NC * _NS
_CH = 80
_RPW = 128
_EPW = _RPW * _CH
_EPAD = _NW * _EPW
_RPT = 632
_ACC_ROWS = _NS * _RPT
_DUMP_ROW = _N + 16
_RP_PAD = 16384


def _matmul(x, w):
    def mm(x_ref, w_ref, o_ref):
        o_ref[...] = jnp.dot(x_ref[...], w_ref[...],
                             preferred_element_type=jnp.float32)
    return pl.pallas_call(
        mm,
        grid=(10,),
        in_specs=[pl.BlockSpec((_N // 10, _D), lambda i: (i, 0)),
                  pl.BlockSpec((_D, _D), lambda i: (0, 0))],
        out_specs=pl.BlockSpec((_N // 10, _D), lambda i: (i, 0)),
        out_shape=jax.ShapeDtypeStruct((_N, _D), jnp.float32),
    )(x, w)


def _add_partials(parts):
    def body(p_ref, o_ref):
        o_ref[...] = p_ref[0] + p_ref[1]
    return pl.pallas_call(
        body,
        grid=(10,),
        in_specs=[pl.BlockSpec((2, _N // 10, _D), lambda i: (0, i, 0))],
        out_specs=pl.BlockSpec((_N // 10, _D), lambda i: (i, 0)),
        out_shape=jax.ShapeDtypeStruct((_N, _D), jnp.float32),
    )(parts)


@functools.partial(
    pl.kernel,
    out_type=jax.ShapeDtypeStruct((_EPAD,), jnp.int32),
    mesh=plsc.VectorSubcoreMesh(core_axis_name="c", subcore_axis_name="s"),
    compiler_params=pltpu.CompilerParams(needs_layout_passes=False),
    scratch_types=[
        pltpu.VMEM((_RP_PAD,), jnp.int32),
        pltpu.VMEM((_EPW,), jnp.int32),
    ],
)
def _build_rids(rp_hbm, out_hbm, rpv, ridv):
    cid = lax.axis_index("c")
    sid = lax.axis_index("s")
    wid = cid * _NS + sid
    ebase = wid * _EPW

    big = jnp.full((16,), jnp.iinfo(jnp.int32).max, jnp.int32)
    def fillmax(i, _):
        rpv[pl.ds(_N + i * 16, 16)] = big
        return 0
    lax.fori_loop(0, (_RP_PAD - _N) // 16, fillmax, 0)
    pltpu.sync_copy(rp_hbm.at[pl.ds(0, _N)], rpv.at[pl.ds(0, _N)])

    iota = lax.iota(jnp.int32, 16)
    dumpv = jnp.full((16,), _N, jnp.int32)
    ev_max = jnp.full((16,), _E, jnp.int32)
    def vec(v, _):
        ev = iota + (ebase + v * 16)
        lo = jnp.zeros((16,), jnp.int32)
        w = _RP_PAD // 2
        while w >= 1:
            cand = lo + jnp.full((16,), w, jnp.int32)
            vals = plsc.load_gather(rpv, [cand])
            lo = jnp.where(vals <= ev, cand, lo)
            w //= 2
        dump = dumpv + (ev & jnp.full((16,), 63, jnp.int32))
        ridv[pl.ds(v * 16, 16)] = jnp.where(ev < ev_max, lo, dump)
        return 0
    lax.fori_loop(0, _EPW // 16, vec, 0)

    pltpu.sync_copy(ridv, out_hbm.at[pl.ds(ebase, _EPW)])


@functools.partial(
    pl.kernel,
    out_type=jax.ShapeDtypeStruct((_NC * _ACC_ROWS, _D), jnp.float32),
    mesh=plsc.VectorSubcoreMesh(core_axis_name="c", subcore_axis_name="s"),
    scratch_types=[
        [pltpu.VMEM((_CH,), jnp.int32)] * 4,
        [pltpu.VMEM((_CH,), jnp.int32)] * 4,
        [pltpu.VMEM((_CH, _D), jnp.float32)] * 4,
        pltpu.VMEM_SHARED((_ACC_ROWS, _D), jnp.float32),
        [pltpu.SemaphoreType.DMA] * 4,
        [pltpu.SemaphoreType.DMA] * 4,
        [pltpu.SemaphoreType.DMA] * 4,
    ],
)
def _spmm_round(xp_hbm, col_hbm, rid_hbm, out_hbm,
                colb, ridb, rows, acc, semg, semi, semsc):
    cid = lax.axis_index("c")
    sid = lax.axis_index("s")
    wid = cid * _NS + sid
    ebase = wid * _EPW

    def zstore(i, _):
        for j in range(_D // 16):
            rows[0][i, pl.ds(j * 16, 16)] = jnp.zeros((16,), jnp.float32)
        return 0
    lax.fori_loop(0, _CH, zstore, 0)

    zbase = sid * _RPT
    def zcopy(k, _):
        pltpu.sync_copy(rows[0], acc.at[pl.ds(zbase + k * _CH, _CH)])
        return 0
    lax.fori_loop(0, _RPT // _CH, zcopy, 0)
    pltpu.sync_copy(rows[0].at[pl.ds(0, _RPT % _CH)],
                    acc.at[pl.ds(zbase + (_RPT // _CH) * _CH, _RPT % _CH)])

    def idxload(k, i):
        off = ebase + k * _CH
        pltpu.async_copy(col_hbm.at[pl.ds(off, _CH)], colb[i], semi[i])
        pltpu.async_copy(rid_hbm.at[pl.ds(off, _CH)], ridb[i], semi[i])

    def idxwait(i):
        pltpu.make_async_copy(col_hbm.at[pl.ds(0, _CH)], colb[i],
                              semi[i]).wait()
        pltpu.make_async_copy(rid_hbm.at[pl.ds(0, _CH)], ridb[i],
                              semi[i]).wait()

    def gather(i):
        pltpu.async_copy(xp_hbm.at[colb[i]], rows[i], semg[i])

    def gwait(i):
        pltpu.make_async_copy(xp_hbm.at[colb[i]], rows[i], semg[i]).wait()

    def scatter(i):
        pltpu.async_copy(rows[i], acc.at[ridb[i]], semsc[i], add=True)

    def scwait(i):
        pltpu.make_async_copy(rows[i], acc.at[ridb[i]], semsc[i]).wait()

    idxload(0, 0)
    idxload(1, 1)

    plsc.subcore_barrier()

    idxwait(0)
    gather(0)

    def step(j, _):
        c0 = 4 * j
        for s in range(4):
            c = c0 + s
            @pl.when(c >= 1)
            def _():
                gwait((s - 1) % 4)
                scatter((s - 1) % 4)

            @pl.when(c >= 2)
            def _():
                scwait((s - 2) % 4)

            @pl.when(c + 2 < _RPW)
            def _():
                idxload(c + 2, (s + 2) % 4)

            @pl.when(c + 1 < _RPW)
            def _():
                idxwait((s + 1) % 4)
                gather((s + 1) % 4)
        return 0
    lax.fori_loop(0, _RPW // 4, step, 0)

    gwait(3)
    scatter(3)
    scwait(2)
    scwait(3)

    plsc.subcore_barrier()

    pltpu.sync_copy(acc.at[pl.ds(zbase, _RPT)],
                    out_hbm.at[pl.ds(cid * _ACC_ROWS + zbase, _RPT)])


def kernel(X, row_pointers, column_index, blockPartition, edgeToColumn,
           edgeToRow, W, attention_w):
    pad = _EPAD - _E
    col_p = jnp.concatenate(
        [column_index, jnp.arange(pad, dtype=jnp.int32) % _N])
    rid_p = _build_rids(row_pointers)
    xp = _matmul(X, W)
    for _ in range(8):
        flat = _spmm_round(xp, col_p, rid_p)
        xp = _add_partials(flat.reshape(_NC, _ACC_ROWS, _D))
    return xp


# speedup vs baseline: 5.8880x; 1.0467x over previous
import functools
import jax
import jax.numpy as jnp
from jax import lax
from jax.experimental import pallas as pl
from jax.experimental.pallas import tpu as pltpu
from jax.experimental.pallas import tpu_sc as plsc

_N = 10000
_E = 320000
_D = 128
_NC = 2
_NS = 16
_NW = _NC * _NS
_CH = 80
_RPW = 128
_EPW = _RPW * _CH
_EPAD = _NW * _EPW
_RPT = 632
_ACC_ROWS = _NS * _RPT
_DUMP_ROW = _N + 16
_RP_PAD = 16384


def _matmul(x, w):
    def mm(x_ref, w_ref, o_ref):
        o_ref[...] = jnp.dot(x_ref[...], w_ref[...],
                             preferred_element_type=jnp.float32)
    return pl.pallas_call(
        mm,
        grid=(10,),
        in_specs=[pl.BlockSpec((_N // 10, _D), lambda i: (i, 0)),
                  pl.BlockSpec((_D, _D), lambda i: (0, 0))],
        out_specs=pl.BlockSpec((_N // 10, _D), lambda i: (i, 0)),
        out_shape=jax.ShapeDtypeStruct((_N, _D), jnp.float32),
    )(x, w)


def _add_partials(parts):
    def body(p_ref, o_ref):
        o_ref[...] = p_ref[0] + p_ref[1]
    return pl.pallas_call(
        body,
        grid=(10,),
        in_specs=[pl.BlockSpec((2, _N // 10, _D), lambda i: (0, i, 0))],
        out_specs=pl.BlockSpec((_N // 10, _D), lambda i: (i, 0)),
        out_shape=jax.ShapeDtypeStruct((_N, _D), jnp.float32),
    )(parts)


@functools.partial(
    pl.kernel,
    out_type=jax.ShapeDtypeStruct((_EPAD,), jnp.int32),
    mesh=plsc.VectorSubcoreMesh(core_axis_name="c", subcore_axis_name="s"),
    compiler_params=pltpu.CompilerParams(needs_layout_passes=False),
    scratch_types=[
        pltpu.VMEM((_RP_PAD,), jnp.int32),
        pltpu.VMEM((_EPW,), jnp.int32),
    ],
)
def _build_rids(rp_hbm, out_hbm, rpv, ridv):
    cid = lax.axis_index("c")
    sid = lax.axis_index("s")
    wid = cid * _NS + sid
    ebase = wid * _EPW

    big = jnp.full((16,), jnp.iinfo(jnp.int32).max, jnp.int32)
    def fillmax(i, _):
        rpv[pl.ds(_N + i * 16, 16)] = big
        return 0
    lax.fori_loop(0, (_RP_PAD - _N) // 16, fillmax, 0)
    pltpu.sync_copy(rp_hbm.at[pl.ds(0, _N)], rpv.at[pl.ds(0, _N)])

    iota = lax.iota(jnp.int32, 16)
    dumpv = jnp.full((16,), _N, jnp.int32)
    ev_max = jnp.full((16,), _E, jnp.int32)
    mask63 = jnp.full((16,), 63, jnp.int32)
    def vec(v, _):
        evs = [iota + (ebase + (4 * v + u) * 16) for u in range(4)]
        los = [jnp.zeros((16,), jnp.int32) for _ in range(4)]
        w = _RP_PAD // 2
        while w >= 1:
            wv = jnp.full((16,), w, jnp.int32)
            cands = [lo + wv for lo in los]
            vals = [plsc.load_gather(rpv, [c]) for c in cands]
            los = [jnp.where(vals[u] <= evs[u], cands[u], los[u])
                   for u in range(4)]
            w //= 2
        for u in range(4):
            dump = dumpv + (evs[u] & mask63)
            ridv[pl.ds((4 * v + u) * 16, 16)] = jnp.where(
                evs[u] < ev_max, los[u], dump)
        return 0
    lax.fori_loop(0, _EPW // 64, vec, 0)

    pltpu.sync_copy(ridv, out_hbm.at[pl.ds(ebase, _EPW)])


@functools.partial(
    pl.kernel,
    out_type=jax.ShapeDtypeStruct((_NC * _ACC_ROWS, _D), jnp.float32),
    mesh=plsc.VectorSubcoreMesh(core_axis_name="c", subcore_axis_name="s"),
    scratch_types=[
        [pltpu.VMEM((_CH,), jnp.int32)] * 4,
        [pltpu.VMEM((_CH,), jnp.int32)] * 4,
        [pltpu.VMEM((_CH, _D), jnp.float32)] * 4,
        pltpu.VMEM_SHARED((_ACC_ROWS, _D), jnp.float32),
        [pltpu.SemaphoreType.DMA] * 4,
        [pltpu.SemaphoreType.DMA] * 4,
        [pltpu.SemaphoreType.DMA] * 4,
    ],
)
def _spmm_round(xp_hbm, col_hbm, rid_hbm, out_hbm,
                colb, ridb, rows, acc, semg, semi, semsc):
    cid = lax.axis_index("c")
    sid = lax.axis_index("s")
    wid = cid * _NS + sid
    ebase = wid * _EPW

    def zstore(i, _):
        for j in range(_D // 16):
            rows[0][i, pl.ds(j * 16, 16)] = jnp.zeros((16,), jnp.float32)
        return 0
    lax.fori_loop(0, _CH, zstore, 0)

    zbase = sid * _RPT
    def zcopy(k, _):
        pltpu.sync_copy(rows[0], acc.at[pl.ds(zbase + k * _CH, _CH)])
        return 0
    lax.fori_loop(0, _RPT // _CH, zcopy, 0)
    pltpu.sync_copy(rows[0].at[pl.ds(0, _RPT % _CH)],
                    acc.at[pl.ds(zbase + (_RPT // _CH) * _CH, _RPT % _CH)])

    def idxload(k, i):
        off = ebase + k * _CH
        pltpu.async_copy(col_hbm.at[pl.ds(off, _CH)], colb[i], semi[i])
        pltpu.async_copy(rid_hbm.at[pl.ds(off, _CH)], ridb[i], semi[i])

    def idxwait(i):
        pltpu.make_async_copy(col_hbm.at[pl.ds(0, _CH)], colb[i],
                              semi[i]).wait()
        pltpu.make_async_copy(rid_hbm.at[pl.ds(0, _CH)], ridb[i],
                              semi[i]).wait()

    def gather(i):
        pltpu.async_copy(xp_hbm.at[colb[i]], rows[i], semg[i])

    def gwait(i):
        pltpu.make_async_copy(xp_hbm.at[colb[i]], rows[i], semg[i]).wait()

    def scatter(i):
        pltpu.async_copy(rows[i], acc.at[ridb[i]], semsc[i], add=True)

    def scwait(i):
        pltpu.make_async_copy(rows[i], acc.at[ridb[i]], semsc[i]).wait()

    idxload(0, 0)
    idxload(1, 1)

    plsc.subcore_barrier()

    idxwait(0)
    gather(0)

    def step(j, _):
        c0 = 4 * j
        for s in range(4):
            c = c0 + s
            @pl.when(c >= 1)
            def _():
                gwait((s - 1) % 4)
                scatter((s - 1) % 4)

            @pl.when(c >= 2)
            def _():
                scwait((s - 2) % 4)

            @pl.when(c + 2 < _RPW)
            def _():
                idxload(c + 2, (s + 2) % 4)

            @pl.when(c + 1 < _RPW)
            def _():
                idxwait((s + 1) % 4)
                gather((s + 1) % 4)
        return 0
    lax.fori_loop(0, _RPW // 4, step, 0)

    gwait(3)
    scatter(3)
    scwait(2)
    scwait(3)

    plsc.subcore_barrier()

    pltpu.sync_copy(acc.at[pl.ds(zbase, _RPT)],
                    out_hbm.at[pl.ds(cid * _ACC_ROWS + zbase, _RPT)])


def kernel(X, row_pointers, column_index, blockPartition, edgeToColumn,
           edgeToRow, W, attention_w):
    pad = _EPAD - _E
    col_p = jnp.concatenate(
        [column_index, jnp.arange(pad, dtype=jnp.int32) % _N])
    rid_p = _build_rids(row_pointers)
    xp = _matmul(X, W)
    for _ in range(8):
        flat = _spmm_round(xp, col_p, rid_p)
        xp = _add_partials(flat.reshape(_NC, _ACC_ROWS, _D))
    return xp


# speedup vs baseline: 6.0310x; 1.0243x over previous
import functools
import jax
import jax.numpy as jnp
from jax import lax
from jax.experimental import pallas as pl
from jax.experimental.pallas import tpu as pltpu
from jax.experimental.pallas import tpu_sc as plsc

_N = 10000
_E = 320000
_D = 128
_NC = 2
_NS = 16
_NW = _NC * _NS
_CH = 80
_RPW = 128
_EPW = _RPW * _CH
_EPAD = _NW * _EPW
_RPT = 632
_ACC_ROWS = _NS * _RPT
_DUMP_ROW = _N + 16
_RP_PAD = 16384


def _matmul(x, w):
    def mm(x_ref, w_ref, o_ref):
        o_ref[...] = jnp.dot(x_ref[...], w_ref[...],
                             preferred_element_type=jnp.float32)
    return pl.pallas_call(
        mm,
        grid=(10,),
        in_specs=[pl.BlockSpec((_N // 10, _D), lambda i: (i, 0)),
                  pl.BlockSpec((_D, _D), lambda i: (0, 0))],
        out_specs=pl.BlockSpec((_N // 10, _D), lambda i: (i, 0)),
        out_shape=jax.ShapeDtypeStruct((_N, _D), jnp.float32),
    )(x, w)


def _add_partials(parts):
    def body(p_ref, o_ref):
        o_ref[...] = p_ref[0] + p_ref[1]
    return pl.pallas_call(
        body,
        grid=(10,),
        in_specs=[pl.BlockSpec((2, _N // 10, _D), lambda i: (0, i, 0))],
        out_specs=pl.BlockSpec((_N // 10, _D), lambda i: (i, 0)),
        out_shape=jax.ShapeDtypeStruct((_N, _D), jnp.float32),
    )(parts)


@functools.partial(
    pl.kernel,
    out_type=jax.ShapeDtypeStruct((_EPAD,), jnp.int32),
    mesh=plsc.VectorSubcoreMesh(core_axis_name="c", subcore_axis_name="s"),
    compiler_params=pltpu.CompilerParams(needs_layout_passes=False),
    scratch_types=[
        pltpu.VMEM((_RP_PAD,), jnp.int32),
        pltpu.VMEM((_EPW,), jnp.int32),
    ],
)
def _build_rids(rp_hbm, out_hbm, rpv, ridv):
    cid = lax.axis_index("c")
    sid = lax.axis_index("s")
    wid = cid * _NS + sid
    ebase = wid * _EPW

    big = jnp.full((16,), jnp.iinfo(jnp.int32).max, jnp.int32)
    def fillmax(i, _):
        rpv[pl.ds(_N + i * 16, 16)] = big
        return 0
    lax.fori_loop(0, (_RP_PAD - _N) // 16, fillmax, 0)
    pltpu.sync_copy(rp_hbm.at[pl.ds(0, _N)], rpv.at[pl.ds(0, _N)])

    iota = lax.iota(jnp.int32, 16)
    dumpv = jnp.full((16,), _N, jnp.int32)
    ev_max = jnp.full((16,), _E, jnp.int32)
    mask63 = jnp.full((16,), 63, jnp.int32)
    iota80 = iota * 80
    def vec(v, _):
        s0s = [(4 * v + u) * 16 for u in range(4)]
        evs = [iota80 + (ebase + (s0 % 128) * 80 + s0 // 128) for s0 in s0s]
        los = [jnp.zeros((16,), jnp.int32) for _ in range(4)]
        w = _RP_PAD // 2
        while w >= 1:
            wv = jnp.full((16,), w, jnp.int32)
            cands = [lo + wv for lo in los]
            vals = [plsc.load_gather(rpv, [c]) for c in cands]
            los = [jnp.where(vals[u] <= evs[u], cands[u], los[u])
                   for u in range(4)]
            w //= 2
        for u in range(4):
            dump = dumpv + (evs[u] & mask63)
            ridv[pl.ds((4 * v + u) * 16, 16)] = jnp.where(
                evs[u] < ev_max, los[u], dump)
        return 0
    lax.fori_loop(0, _EPW // 64, vec, 0)

    pltpu.sync_copy(ridv, out_hbm.at[pl.ds(ebase, _EPW)])


@functools.partial(
    pl.kernel,
    out_type=jax.ShapeDtypeStruct((_NC * _ACC_ROWS, _D), jnp.float32),
    mesh=plsc.VectorSubcoreMesh(core_axis_name="c", subcore_axis_name="s"),
    scratch_types=[
        [pltpu.VMEM((_CH,), jnp.int32)] * 4,
        [pltpu.VMEM((_CH,), jnp.int32)] * 4,
        [pltpu.VMEM((_CH, _D), jnp.float32)] * 4,
        pltpu.VMEM_SHARED((_ACC_ROWS, _D), jnp.float32),
        [pltpu.SemaphoreType.DMA] * 4,
        [pltpu.SemaphoreType.DMA] * 4,
        [pltpu.SemaphoreType.DMA] * 4,
    ],
)
def _spmm_round(xp_hbm, col_hbm, rid_hbm, out_hbm,
                colb, ridb, rows, acc, semg, semi, semsc):
    cid = lax.axis_index("c")
    sid = lax.axis_index("s")
    wid = cid * _NS + sid
    ebase = wid * _EPW

    def zstore(i, _):
        for j in range(_D // 16):
            rows[0][i, pl.ds(j * 16, 16)] = jnp.zeros((16,), jnp.float32)
        return 0
    lax.fori_loop(0, _CH, zstore, 0)

    zbase = sid * _RPT
    def zcopy(k, _):
        pltpu.sync_copy(rows[0], acc.at[pl.ds(zbase + k * _CH, _CH)])
        return 0
    lax.fori_loop(0, _RPT // _CH, zcopy, 0)
    pltpu.sync_copy(rows[0].at[pl.ds(0, _RPT % _CH)],
                    acc.at[pl.ds(zbase + (_RPT // _CH) * _CH, _RPT % _CH)])

    def idxload(k, i):
        off = ebase + k * _CH
        pltpu.async_copy(col_hbm.at[pl.ds(off, _CH)], colb[i], semi[i])
        pltpu.async_copy(rid_hbm.at[pl.ds(off, _CH)], ridb[i], semi[i])

    def idxwait(i):
        pltpu.make_async_copy(col_hbm.at[pl.ds(0, _CH)], colb[i],
                              semi[i]).wait()
        pltpu.make_async_copy(rid_hbm.at[pl.ds(0, _CH)], ridb[i],
                              semi[i]).wait()

    def gather(i):
        pltpu.async_copy(xp_hbm.at[colb[i]], rows[i], semg[i])

    def gwait(i):
        pltpu.make_async_copy(xp_hbm.at[colb[i]], rows[i], semg[i]).wait()

    def scatter(i):
        pltpu.async_copy(rows[i], acc.at[ridb[i]], semsc[i], add=True)

    def scwait(i):
        pltpu.make_async_copy(rows[i], acc.at[ridb[i]], semsc[i]).wait()

    idxload(0, 0)
    idxload(1, 1)

    plsc.subcore_barrier()

    idxwait(0)
    gather(0)

    def step(j, _):
        c0 = 4 * j
        for s in range(4):
            c = c0 + s
            @pl.when(c >= 1)
            def _():
                gwait((s - 1) % 4)
                scatter((s - 1) % 4)

            @pl.when(c >= 2)
            def _():
                scwait((s - 2) % 4)

            @pl.when(c + 2 < _RPW)
            def _():
                idxload(c + 2, (s + 2) % 4)

            @pl.when(c + 1 < _RPW)
            def _():
                idxwait((s + 1) % 4)
                gather((s + 1) % 4)
        return 0
    lax.fori_loop(0, _RPW // 4, step, 0)

    gwait(3)
    scatter(3)
    scwait(2)
    scwait(3)

    plsc.subcore_barrier()

    pltpu.sync_copy(acc.at[pl.ds(zbase, _RPT)],
                    out_hbm.at[pl.ds(cid * _ACC_ROWS + zbase, _RPT)])


def kernel(X, row_pointers, column_index, blockPartition, edgeToColumn,
           edgeToRow, W, attention_w):
    pad = _EPAD - _E
    col_p = jnp.concatenate(
        [column_index, jnp.arange(pad, dtype=jnp.int32) % _N])
    col_p = col_p.reshape(_NW, _RPW, _CH).transpose(0, 2, 1).reshape(-1)
    rid_p = _build_rids(row_pointers)
    xp = _matmul(X, W)
    for _ in range(8):
        flat = _spmm_round(xp, col_p, rid_p)
        xp = _add_partials(flat.reshape(_NC, _ACC_ROWS, _D))
    return xp
